# Initial kernel scaffold; baseline (speedup 1.0000x reference)
import jax
import jax.numpy as jnp
from jax.experimental import pallas as pl


def kernel(d, Angles, id_expand_kj):
    raise NotImplementedError("write your pallas kernel here")


# speedup vs baseline: 5.8757x; 5.8757x over previous
import functools
import math

import numpy as np
import jax
import jax.numpy as jnp
from jax import lax
from jax.experimental import pallas as pl
from jax.experimental.pallas import tpu as pltpu
from jax.experimental.pallas import tpu_sc as plsc

NUM_SPHERICAL = 7
NUM_RADIAL = 6
NUM_BASIS = NUM_SPHERICAL * NUM_RADIAL
CUTOFF = 5.0
ENVELOPE_EXPONENT = 5
N_EDGES = 160000
N_TRIPLETS = 640000


def _jn_host(r, n):
    r = np.asarray(r, dtype=np.float64)
    j0 = np.sin(r) / r
    if n == 0:
        return j0
    j1 = np.sin(r) / r**2 - np.cos(r) / r
    if n == 1:
        return j1
    jm1, jc = j0, j1
    for l in range(1, n):
        jn_ = (2 * l + 1) / r * jc - jm1
        jm1, jc = jc, jn_
    return jc


def _bessel_zeros(n, k):
    zerosj = np.zeros((n, k), dtype=np.float64)
    zerosj[0] = np.arange(1, k + 1) * np.pi
    points = np.arange(1, k + n) * np.pi
    racines = np.zeros(k + n - 1, dtype=np.float64)
    for i in range(1, n):
        for j in range(k + n - 1 - i):
            a, b = points[j], points[j + 1]
            fa = _jn_host(a, i)
            for _ in range(100):
                m = 0.5 * (a + b)
                fm = _jn_host(m, i)
                if fa * fm <= 0:
                    b = m
                else:
                    a, fa = m, fm
            racines[j] = 0.5 * (a + b)
        points = racines.copy()
        zerosj[i][:k] = racines[:k]
    return zerosj


_ZEROS = _bessel_zeros(NUM_SPHERICAL, NUM_RADIAL)
_NORM = np.zeros((NUM_SPHERICAL, NUM_RADIAL), dtype=np.float64)
for _l in range(NUM_SPHERICAL):
    for _n in range(NUM_RADIAL):
        _NORM[_l, _n] = 1.0 / np.sqrt(0.5 * _jn_host(_ZEROS[_l, _n], _l + 1) ** 2)

_Z_ROW = np.asarray(
    [float(_ZEROS[c // NUM_RADIAL, c % NUM_RADIAL]) for c in range(NUM_BASIS)],
    dtype=np.float32,
).reshape(1, NUM_BASIS)
_NORM_ROW = np.asarray(
    [float(_NORM[c // NUM_RADIAL, c % NUM_RADIAL]) for c in range(NUM_BASIS)],
    dtype=np.float32,
).reshape(1, NUM_BASIS)
_L_ROW = np.asarray(
    [c // NUM_RADIAL for c in range(NUM_BASIS)], dtype=np.int32
).reshape(1, NUM_BASIS)

_PREF = [float(math.sqrt((2 * l + 1) / (4 * math.pi))) for l in range(NUM_SPHERICAL)]


_TC_GRID = 25
_D_BLK = N_EDGES // _TC_GRID
_ANG_COLS = 128
_ANG_ROWS = N_TRIPLETS // _ANG_COLS
_ANG_BLK = _ANG_ROWS // _TC_GRID


def _tc_body(d_ref, ang_ref, zrow_ref, normrow_ref, lrow_ref, table_ref, ct_ref):
    x = d_ref[...] * (1.0 / CUTOFF)
    zrow = zrow_ref[...]
    X = x * zrow
    s = jnp.sin(X)
    c = jnp.cos(X)
    j0 = s / X
    j1 = s / (X * X) - c / X
    lrow = lrow_ref[...]
    res = jnp.where(lrow == 0, j0, j1)
    jm1, jc = j0, j1
    for ll in range(1, NUM_SPHERICAL - 1):
        jn_ = (2 * ll + 1) / X * jc - jm1
        res = jnp.where(lrow == ll + 1, jn_, res)
        jm1, jc = jc, jn_
    p = ENVELOPE_EXPONENT + 1
    a = -(p + 1) * (p + 2) / 2
    b = p * (p + 2)
    cc = -p * (p + 1) / 2
    env = 1.0 / x + a * x ** (p - 1) + b * x ** p + cc * x ** (p + 1)
    env = jnp.where(x < 1, env, jnp.zeros_like(x))
    table_ref[...] = env * (normrow_ref[...] * res)
    ct_ref[...] = jnp.cos(ang_ref[...])


_tc_call = pl.pallas_call(
    _tc_body,
    grid=(_TC_GRID,),
    in_specs=[
        pl.BlockSpec((_D_BLK, 1), lambda i: (i, 0)),
        pl.BlockSpec((_ANG_BLK, _ANG_COLS), lambda i: (i, 0)),
        pl.BlockSpec((1, NUM_BASIS), lambda i: (0, 0)),
        pl.BlockSpec((1, NUM_BASIS), lambda i: (0, 0)),
        pl.BlockSpec((1, NUM_BASIS), lambda i: (0, 0)),
    ],
    out_specs=[
        pl.BlockSpec((_D_BLK, NUM_BASIS), lambda i: (i, 0)),
        pl.BlockSpec((_ANG_BLK, _ANG_COLS), lambda i: (i, 0)),
    ],
    out_shape=[
        jax.ShapeDtypeStruct((N_EDGES, NUM_BASIS), jnp.float32),
        jax.ShapeDtypeStruct((_ANG_ROWS, _ANG_COLS), jnp.float32),
    ],
)


_NC = 2
_NS = 16
_NW = _NC * _NS
_PER_W = N_TRIPLETS // _NW
_BC = 1000
_NCHUNK = _PER_W // _BC
_LANES = 16
_GROUP_ROWS = 8
_VPG = _GROUP_ROWS * NUM_BASIS // _LANES
_NG = _BC // _GROUP_ROWS

_pos = np.arange(_VPG * _LANES, dtype=np.int32)
_TOF_NP = (_pos // NUM_BASIS).reshape(_VPG, _LANES)
_COL_NP = (_pos % NUM_BASIS).reshape(_VPG, _LANES)
_LL_NP = (_COL_NP // NUM_RADIAL).astype(np.int32)

@functools.lru_cache(maxsize=1)
def _make_sc_gather_mul():
  sc_mesh = plsc.VectorSubcoreMesh(core_axis_name="c", subcore_axis_name="s")

  @functools.partial(
      pl.kernel,
      mesh=sc_mesh,
      out_type=jax.ShapeDtypeStruct((N_TRIPLETS * NUM_BASIS,), jnp.float32),
      compiler_params=pltpu.CompilerParams(
          needs_layout_passes=False, use_tc_tiling_on_sc=False),
      scratch_types=[
          pltpu.VMEM((_BC,), jnp.int32),
          pltpu.VMEM((_BC,), jnp.float32),
          pltpu.VMEM((NUM_SPHERICAL, _BC), jnp.float32),
          pltpu.VMEM((_BC, NUM_BASIS), jnp.float32),
          pltpu.VMEM((_BC * NUM_BASIS,), jnp.float32),
          pltpu.VMEM((_VPG, _LANES), jnp.int32),
          pltpu.VMEM((_VPG, _LANES), jnp.int32),
          pltpu.VMEM((_VPG, _LANES), jnp.int32),
          pltpu.SemaphoreType.DMA,
      ],
  )
  def _sc_gather_mul(table_hbm, idx_hbm, ct_hbm, tof_hbm, col_hbm, ll_hbm,
                     out_hbm, idx_v, ct_v, cT_v, rows_v, out_v,
                     tof_v, col_v, ll_v, sem):
    wid = lax.axis_index("s") * _NC + lax.axis_index("c")
    pltpu.sync_copy(tof_hbm, tof_v)
    pltpu.sync_copy(col_hbm, col_v)
    pltpu.sync_copy(ll_hbm, ll_v)

    def chunk_body(ci, carry):
        base = wid * _PER_W + ci * _BC
        pltpu.sync_copy(idx_hbm.at[pl.ds(base, _BC)], idx_v)
        pltpu.sync_copy(ct_hbm.at[pl.ds(base, _BC)], ct_v)
        gat = pltpu.async_copy(table_hbm.at[idx_v], rows_v, sem)

        def leg_body(tt, c2):
            xv = ct_v[pl.ds(tt * _LANES, _LANES)]
            cT_v[0, pl.ds(tt * _LANES, _LANES)] = jnp.full(
                (_LANES,), _PREF[0], jnp.float32)
            p_prev = jnp.ones((_LANES,), jnp.float32)
            p_cur = xv
            cT_v[1, pl.ds(tt * _LANES, _LANES)] = _PREF[1] * p_cur
            for l in range(1, NUM_SPHERICAL - 1):
                p_next = ((2 * l + 1) * xv * p_cur - l * p_prev) / (l + 1)
                cT_v[l + 1, pl.ds(tt * _LANES, _LANES)] = _PREF[l + 1] * p_next
                p_prev, p_cur = p_cur, p_next
            return c2

        lax.fori_loop(0, _BC // _LANES, leg_body, 0)
        gat.wait()

        def grp_body(g, c2):
            t0 = g * _GROUP_ROWS
            for v in range(_VPG):
                t16 = tof_v[v, :] + t0
                rbf = plsc.load_gather(rows_v, [t16, col_v[v, :]])
                w = plsc.load_gather(cT_v, [ll_v[v, :], t16])
                out_v[pl.ds(g * (_GROUP_ROWS * NUM_BASIS) + v * _LANES,
                            _LANES)] = rbf * w
            return c2

        lax.fori_loop(0, _NG, grp_body, 0)
        pltpu.sync_copy(out_v, out_hbm.at[pl.ds(base * NUM_BASIS,
                                                _BC * NUM_BASIS)])
        return carry

    lax.fori_loop(0, _NCHUNK, chunk_body, 0)

  return _sc_gather_mul


def kernel(d, Angles, id_expand_kj):
    d2 = d.reshape(N_EDGES, 1)
    ang2 = Angles.reshape(_ANG_ROWS, _ANG_COLS)
    table, ct2 = _tc_call(d2, ang2, jnp.asarray(_Z_ROW),
                          jnp.asarray(_NORM_ROW), jnp.asarray(_L_ROW))
    ct = ct2.reshape(N_TRIPLETS)
    out_flat = _make_sc_gather_mul()(
        table, id_expand_kj, ct,
        jnp.asarray(_TOF_NP), jnp.asarray(_COL_NP), jnp.asarray(_LL_NP))
    return out_flat.reshape(N_TRIPLETS, NUM_BASIS)


# speedup vs baseline: 8.7618x; 1.4912x over previous
import functools
import math

import numpy as np
import jax
import jax.numpy as jnp
from jax import lax
from jax.experimental import pallas as pl
from jax.experimental.pallas import tpu as pltpu
from jax.experimental.pallas import tpu_sc as plsc

NUM_SPHERICAL = 7
NUM_RADIAL = 6
NUM_BASIS = NUM_SPHERICAL * NUM_RADIAL
PAD_BASIS = 48
CUTOFF = 5.0
ENVELOPE_EXPONENT = 5
N_EDGES = 160000
N_TRIPLETS = 640000


def _jn_host(r, n):
    r = np.asarray(r, dtype=np.float64)
    j0 = np.sin(r) / r
    if n == 0:
        return j0
    j1 = np.sin(r) / r**2 - np.cos(r) / r
    if n == 1:
        return j1
    jm1, jc = j0, j1
    for l in range(1, n):
        jn_ = (2 * l + 1) / r * jc - jm1
        jm1, jc = jc, jn_
    return jc


def _bessel_zeros(n, k):
    zerosj = np.zeros((n, k), dtype=np.float64)
    zerosj[0] = np.arange(1, k + 1) * np.pi
    points = np.arange(1, k + n) * np.pi
    racines = np.zeros(k + n - 1, dtype=np.float64)
    for i in range(1, n):
        for j in range(k + n - 1 - i):
            a, b = points[j], points[j + 1]
            fa = _jn_host(a, i)
            for _ in range(100):
                m = 0.5 * (a + b)
                fm = _jn_host(m, i)
                if fa * fm <= 0:
                    b = m
                else:
                    a, fa = m, fm
            racines[j] = 0.5 * (a + b)
        points = racines.copy()
        zerosj[i][:k] = racines[:k]
    return zerosj


_ZEROS = _bessel_zeros(NUM_SPHERICAL, NUM_RADIAL)
_NORM = np.zeros((NUM_SPHERICAL, NUM_RADIAL), dtype=np.float64)
for _l in range(NUM_SPHERICAL):
    for _n in range(NUM_RADIAL):
        _NORM[_l, _n] = 1.0 / np.sqrt(0.5 * _jn_host(_ZEROS[_l, _n], _l + 1) ** 2)

_Z_ROW = np.ones((1, PAD_BASIS), dtype=np.float32)
_NORM_ROW = np.zeros((1, PAD_BASIS), dtype=np.float32)
_L_ROW = np.zeros((1, PAD_BASIS), dtype=np.int32)
for _c in range(NUM_BASIS):
    _Z_ROW[0, _c] = float(_ZEROS[_c // NUM_RADIAL, _c % NUM_RADIAL])
    _NORM_ROW[0, _c] = float(_NORM[_c // NUM_RADIAL, _c % NUM_RADIAL])
    _L_ROW[0, _c] = _c // NUM_RADIAL

_PREF = [float(math.sqrt((2 * l + 1) / (4 * math.pi))) for l in range(NUM_SPHERICAL)]


_TC_GRID = 25
_D_BLK = N_EDGES // _TC_GRID
_ANG_COLS = 128
_ANG_ROWS = N_TRIPLETS // _ANG_COLS
_ANG_BLK = _ANG_ROWS // _TC_GRID


def _tc_body(d_ref, ang_ref, zrow_ref, normrow_ref, lrow_ref, table_ref, ct_ref):
    x = d_ref[...] * (1.0 / CUTOFF)
    zrow = zrow_ref[...]
    X = x * zrow
    s = jnp.sin(X)
    c = jnp.cos(X)
    j0 = s / X
    j1 = s / (X * X) - c / X
    lrow = lrow_ref[...]
    res = jnp.where(lrow == 0, j0, j1)
    jm1, jc = j0, j1
    for ll in range(1, NUM_SPHERICAL - 1):
        jn_ = (2 * ll + 1) / X * jc - jm1
        res = jnp.where(lrow == ll + 1, jn_, res)
        jm1, jc = jc, jn_
    p = ENVELOPE_EXPONENT + 1
    a = -(p + 1) * (p + 2) / 2
    b = p * (p + 2)
    cc = -p * (p + 1) / 2
    env = 1.0 / x + a * x ** (p - 1) + b * x ** p + cc * x ** (p + 1)
    env = jnp.where(x < 1, env, jnp.zeros_like(x))
    table_ref[...] = env * (normrow_ref[...] * res)
    ct_ref[...] = jnp.cos(ang_ref[...])


_tc_call = pl.pallas_call(
    _tc_body,
    grid=(_TC_GRID,),
    in_specs=[
        pl.BlockSpec((_D_BLK, 1), lambda i: (i, 0)),
        pl.BlockSpec((_ANG_BLK, _ANG_COLS), lambda i: (i, 0)),
        pl.BlockSpec((1, PAD_BASIS), lambda i: (0, 0)),
        pl.BlockSpec((1, PAD_BASIS), lambda i: (0, 0)),
        pl.BlockSpec((1, PAD_BASIS), lambda i: (0, 0)),
    ],
    out_specs=[
        pl.BlockSpec((_D_BLK, PAD_BASIS), lambda i: (i, 0)),
        pl.BlockSpec((_ANG_BLK, _ANG_COLS), lambda i: (i, 0)),
    ],
    out_shape=[
        jax.ShapeDtypeStruct((N_EDGES, PAD_BASIS), jnp.float32),
        jax.ShapeDtypeStruct((_ANG_ROWS, _ANG_COLS), jnp.float32),
    ],
)


_NC = 2
_NS = 16
_NW = _NC * _NS
_PER_W = N_TRIPLETS // _NW
_BC = 400
_NCHUNK = _PER_W // _BC
_LANES = 16
_OUT_CHUNK = _BC * NUM_BASIS
_OFFS = (0, 16, 26)


@functools.lru_cache(maxsize=1)
def _make_sc_gather_mul():
  sc_mesh = plsc.VectorSubcoreMesh(core_axis_name="c", subcore_axis_name="s")

  @functools.partial(
      pl.kernel,
      mesh=sc_mesh,
      out_type=jax.ShapeDtypeStruct((N_TRIPLETS * NUM_BASIS,), jnp.float32),
      compiler_params=pltpu.CompilerParams(
          needs_layout_passes=False, use_tc_tiling_on_sc=False),
      scratch_types=[
          pltpu.VMEM((_BC, PAD_BASIS), jnp.float32),
          pltpu.VMEM((_BC, PAD_BASIS), jnp.float32),
          pltpu.VMEM((_OUT_CHUNK,), jnp.float32),
          pltpu.VMEM((_OUT_CHUNK,), jnp.float32),
          pltpu.VMEM((_BC,), jnp.int32),
          pltpu.VMEM((_BC,), jnp.int32),
          pltpu.VMEM((_BC,), jnp.float32),
          pltpu.VMEM((_BC,), jnp.float32),
          pltpu.VMEM((NUM_SPHERICAL * _BC,), jnp.float32),
          pltpu.SemaphoreType.DMA,
          pltpu.SemaphoreType.DMA,
          pltpu.SemaphoreType.DMA,
          pltpu.SemaphoreType.DMA,
          pltpu.SemaphoreType.DMA,
          pltpu.SemaphoreType.DMA,
          pltpu.SemaphoreType.DMA,
          pltpu.SemaphoreType.DMA,
      ],
  )
  def _sc_gather_mul(table_hbm, idx_hbm, ct_hbm, out_hbm,
                     rows0, rows1, out0, out1, idx0, idx1, ctb0, ctb1, cT,
                     gsem0, gsem1, osem0, osem1, isem0, isem1, csem0, csem1):
    wid = lax.axis_index("s") * _NC + lax.axis_index("c")
    tile_base = wid * _PER_W
    rows = (rows0, rows1)
    outv = (out0, out1)
    idxb = (idx0, idx1)
    ctb = (ctb0, ctb1)
    gsem = (gsem0, gsem1)
    osem = (osem0, osem1)
    isem = (isem0, isem1)
    csem = (csem0, csem1)

    iota = lax.broadcasted_iota(jnp.int32, (_LANES,), 0)
    wq = [((off + iota) // NUM_RADIAL) * _BC for off in _OFFS]

    def start_idx_ct(c, b):
        base = tile_base + c * _BC
        pltpu.async_copy(idx_hbm.at[pl.ds(base, _BC)], idxb[b], isem[b])
        pltpu.async_copy(ct_hbm.at[pl.ds(base, _BC)], ctb[b], csem[b])

    def wait_idx(b):
        pltpu.make_async_copy(
            idx_hbm.at[pl.ds(0, _BC)], idxb[b], isem[b]).wait()

    def wait_ct(b):
        pltpu.make_async_copy(
            ct_hbm.at[pl.ds(0, _BC)], ctb[b], csem[b]).wait()

    def start_gather(b):
        pltpu.async_copy(table_hbm.at[idxb[b]], rows[b], gsem[b])

    def wait_gather(b):
        pltpu.make_async_copy(
            table_hbm.at[pl.ds(0, _BC)], rows[b], gsem[b]).wait()

    def start_out(c, b):
        base = (tile_base + c * _BC) * NUM_BASIS
        pltpu.async_copy(outv[b], out_hbm.at[pl.ds(base, _OUT_CHUNK)], osem[b])

    def wait_out(b):
        pltpu.make_async_copy(
            outv[b], out_hbm.at[pl.ds(0, _OUT_CHUNK)], osem[b]).wait()

    def legendre(b):
        @plsc.parallel_loop(0, _BC // _LANES, 1, unroll=2)
        def _leg(tt):
            xv = ctb[b][pl.ds(tt * _LANES, _LANES)]
            cT[pl.ds(tt * _LANES, _LANES)] = jnp.full(
                (_LANES,), _PREF[0], jnp.float32)
            p_prev = jnp.ones((_LANES,), jnp.float32)
            p_cur = xv
            cT[pl.ds(_BC + tt * _LANES, _LANES)] = _PREF[1] * p_cur
            for l in range(1, NUM_SPHERICAL - 1):
                p_next = ((2 * l + 1) * xv * p_cur - l * p_prev) / (l + 1)
                cT[pl.ds((l + 1) * _BC + tt * _LANES, _LANES)] = (
                    _PREF[l + 1] * p_next)
                p_prev, p_cur = p_cur, p_next

    def multiply(b):
        @plsc.parallel_loop(0, _BC, 1, unroll=2)
        def _mul(t):
            t42 = t * NUM_BASIS
            for k, off in enumerate(_OFFS):
                rbf = rows[b][t, pl.ds(off, _LANES)]
                w = plsc.load_gather(cT, [wq[k] + t])
                outv[b][pl.ds(t42 + off, _LANES)] = rbf * w

    start_idx_ct(0, 0)
    start_idx_ct(1, 1)
    wait_idx(0)
    start_gather(0)

    def chunk(c, b, *, prefetch, next_gather, outwait):
        wait_gather(b)
        wait_ct(b)
        legendre(b)
        if prefetch is not None:
            start_idx_ct(prefetch, b)
        if next_gather:
            wait_idx(1 - b)
            start_gather(1 - b)
        if outwait:
            wait_out(b)
        multiply(b)
        start_out(c, b)

    chunk(0, 0, prefetch=2, next_gather=True, outwait=False)
    chunk(1, 1, prefetch=3, next_gather=True, outwait=False)

    @pl.loop(1, _NCHUNK // 2 - 1)
    def _pair(j):
        c = j * 2
        chunk(c, 0, prefetch=c + 2, next_gather=True, outwait=True)
        chunk(c + 1, 1, prefetch=c + 3, next_gather=True, outwait=True)

    chunk(_NCHUNK - 2, 0, prefetch=None, next_gather=True, outwait=True)
    chunk(_NCHUNK - 1, 1, prefetch=None, next_gather=False, outwait=True)
    wait_out(0)
    wait_out(1)

  return _sc_gather_mul


def kernel(d, Angles, id_expand_kj):
    d2 = d.reshape(N_EDGES, 1)
    ang2 = Angles.reshape(_ANG_ROWS, _ANG_COLS)
    table, ct2 = _tc_call(d2, ang2, jnp.asarray(_Z_ROW),
                          jnp.asarray(_NORM_ROW), jnp.asarray(_L_ROW))
    ct = ct2.reshape(N_TRIPLETS)
    out_flat = _make_sc_gather_mul()(table, id_expand_kj, ct)
    return out_flat.reshape(N_TRIPLETS, NUM_BASIS)
